# baseline (speedup 1.0000x reference)
import jax
import jax.numpy as jnp
from jax.experimental import pallas as pl
from jax.experimental.pallas import tpu as pltpu

_K = 128
_ROWS = 128
_N = 8192
_NBLK = _N // _K
_DISTS = (64, 32, 16, 8, 4, 2, 1)


def _cmp_ex(w, d, keep_max):
    up = (jax.lax.broadcasted_iota(jnp.int32, (1, 1, _K), 2) & d) == 0
    partner = jnp.where(up, pltpu.roll(w, _K - d, 2), pltpu.roll(w, d, 2))
    return jnp.where(keep_max, jnp.maximum(w, partner),
                     jnp.minimum(w, partner))


def _topk_body(x_ref, o_ref):
    w = jnp.stack(
        [x_ref[:, b * _K:(b + 1) * _K] for b in range(_NBLK)], axis=0
    )

    lane = jax.lax.broadcasted_iota(jnp.int32, (1, 1, _K), 2)
    up_of = {d: (lane & d) == 0 for d in _DISTS}

    desc = jax.lax.broadcasted_iota(jnp.int32, (_NBLK, 1, 1), 0) < (_NBLK // 2)
    m = 2
    while m <= _K:
        asc = ((lane & m) == 0) != desc
        d = m // 2
        while d >= 1:
            w = _cmp_ex(w, d, up_of[d] != asc)
            d //= 2
        m *= 2

    nblk = _NBLK
    while nblk > 1:
        nblk //= 2
        w = jnp.maximum(w[:nblk], w[nblk:])
        if nblk > 1:
            asc = (jax.lax.broadcasted_iota(jnp.int32, (nblk, 1, 1), 0)
                   >= (nblk // 2))
            keep = {d: up_of[d] != asc for d in _DISTS}
        else:
            keep = {d: up_of[d] for d in _DISTS}
        for d in _DISTS:
            w = _cmp_ex(w, d, keep[d])

    o_ref[...] = w[0]


def kernel(x):
    return pl.pallas_call(
        _topk_body,
        out_shape=jax.ShapeDtypeStruct((_ROWS, _K), jnp.float32),
        in_specs=[pl.BlockSpec((_ROWS, _N), lambda: (0, 0))],
        out_specs=pl.BlockSpec((_ROWS, _K), lambda: (0, 0)),
    )(x)


# speedup vs baseline: 1.7224x; 1.7224x over previous
import jax
import jax.numpy as jnp
from jax.experimental import pallas as pl

_K = 128
_ROWS = 128
_N = 8192
_NBLK = _N // _K
_DISTS = (64, 32, 16, 8, 4, 2, 1)


def _cmp_ex(w, d, keep_max):
    perm = jax.lax.broadcasted_iota(jnp.int32, w.shape, 1) ^ d
    partner = jnp.take_along_axis(w, perm, axis=1)
    return jnp.where(keep_max, jnp.maximum(w, partner),
                     jnp.minimum(w, partner))


def _topk_body(x_ref, o_ref):
    w = jnp.stack(
        [x_ref[:, b * _K:(b + 1) * _K] for b in range(_NBLK)], axis=1
    ).reshape(_ROWS * _NBLK, _K)

    lane = jax.lax.broadcasted_iota(jnp.int32, (1, _K), 1)
    up_of = {d: (lane & d) == 0 for d in _DISTS}

    nblk = _NBLK
    blk = jax.lax.broadcasted_iota(jnp.int32, (_ROWS * nblk, 1), 0)
    desc = (blk & (nblk // 2)) == 0
    m = 2
    while m <= _K:
        asc = ((lane & m) == 0) != desc
        d = m // 2
        while d >= 1:
            w = _cmp_ex(w, d, up_of[d] != asc)
            d //= 2
        m *= 2

    while nblk > 1:
        nblk //= 2
        v = w.reshape(_ROWS, 2 * nblk, _K)
        w = jnp.maximum(v[:, :nblk], v[:, nblk:]).reshape(_ROWS * nblk, _K)
        if nblk > 1:
            blk = jax.lax.broadcasted_iota(jnp.int32, (_ROWS * nblk, 1), 0)
            asc = (blk & (nblk // 2)) != 0
            keep = {d: up_of[d] != asc for d in _DISTS}
        else:
            keep = {d: up_of[d] for d in _DISTS}
        for d in _DISTS:
            w = _cmp_ex(w, d, keep[d])

    o_ref[...] = w


def kernel(x):
    return pl.pallas_call(
        _topk_body,
        out_shape=jax.ShapeDtypeStruct((_ROWS, _K), jnp.float32),
        in_specs=[pl.BlockSpec((_ROWS, _N), lambda: (0, 0))],
        out_specs=pl.BlockSpec((_ROWS, _K), lambda: (0, 0)),
    )(x)
